# Initial kernel scaffold; baseline (speedup 1.0000x reference)
import functools

import jax
import jax.numpy as jnp
from jax import lax
from jax.experimental import pallas as pl
from jax.experimental.pallas import tpu as pltpu
from jax.experimental.pallas import tpu_sc as plsc

CH = 100


def _make_aggregate(N, E, D):
    Dh = D // 2
    n_sub = 16
    epw = E // n_sub
    nch = epw // CH
    rpt = N // n_sub
    assert epw % CH == 0 and nch % 2 == 0 and N % n_sub == 0

    mesh = plsc.VectorSubcoreMesh(core_axis_name="c", subcore_axis_name="s")

    @functools.partial(
        pl.kernel,
        out_type=jax.ShapeDtypeStruct((2 * N, Dh), jnp.float32),
        mesh=mesh,
        scratch_types=[
            pltpu.VMEM((nch, CH), jnp.int32),
            pltpu.VMEM((CH, Dh), jnp.float32),
            pltpu.VMEM((CH, Dh), jnp.float32),
            pltpu.VMEM_SHARED((N, Dh), jnp.float32),
            pltpu.SemaphoreType.DMA,
            pltpu.SemaphoreType.DMA,
        ],
    )
    def agg(ea_hbm, dst_hbm, zeros_hbm, out_hbm, dst_v, rb0, rb1, acc, sem0, sem1):
        c = lax.axis_index("c")
        s = lax.axis_index("s")
        e_base = s * epw
        col = c * Dh

        pltpu.sync_copy(dst_hbm.at[pl.ds(s * nch, nch), :], dst_v)
        pltpu.sync_copy(zeros_hbm, acc.at[pl.ds(s * rpt, rpt), :])
        plsc.subcore_barrier()

        def rows_src(k):
            return ea_hbm.at[pl.ds(e_base + k * CH, CH), pl.ds(col, Dh)]

        pltpu.async_copy(rows_src(0), rb0, sem0)

        def body(j, carry):
            k = 2 * j
            pltpu.async_copy(rows_src(k + 1), rb1, sem1)
            pltpu.make_async_copy(rows_src(k), rb0, sem0).wait()
            pltpu.sync_copy(rb0, acc.at[dst_v.at[k]], add=True)

            @pl.when(j < nch // 2 - 1)
            def _():
                pltpu.async_copy(rows_src(k + 2), rb0, sem0)

            pltpu.make_async_copy(rows_src(k + 1), rb1, sem1).wait()
            pltpu.sync_copy(rb1, acc.at[dst_v.at[k + 1]], add=True)
            return carry

        lax.fori_loop(0, nch // 2, body, 0)
        plsc.subcore_barrier()

        pltpu.sync_copy(
            acc.at[pl.ds(s * rpt, rpt), :],
            out_hbm.at[pl.ds(c * N + s * rpt, rpt), :],
        )

    return agg


def _mlp_body(nf, a0, a1, w1, w2, b1, g, b, b2, out, *, D, Dh):
    x = nf[...]
    h = jnp.dot(x, w1[0:D, :], preferred_element_type=jnp.float32)
    h = h + jnp.dot(a0[...], w1[D:D + Dh, :], preferred_element_type=jnp.float32)
    h = h + jnp.dot(a1[...], w1[D + Dh:, :], preferred_element_type=jnp.float32)
    h = h + b1[...]
    mu = jnp.mean(h, axis=-1, keepdims=True)
    var = jnp.mean((h - mu) ** 2, axis=-1, keepdims=True)
    hn = (h - mu) * lax.rsqrt(var + 1e-5) * g[...] + b[...]
    hs = hn * jax.nn.sigmoid(hn)
    out[...] = x + jnp.dot(hs, w2[...], preferred_element_type=jnp.float32) + b2[...]


def kernel(node_feat, edge_attr, edge_index, num_nodes, W1, b1, ln_g, ln_b, W2, b2):
    N, D = node_feat.shape
    E = edge_attr.shape[0]
    IN, H = W1.shape
    Dh = D // 2

    dst3 = edge_index[1].reshape(E // CH, CH)
    zeros = jnp.zeros((N // 16, Dh), jnp.float32)
    agg2 = _make_aggregate(N, E, D)(edge_attr, dst3, zeros)
    a0, a1 = agg2[:N], agg2[N:]

    R = 1000
    grid = (N // R,)
    out = pl.pallas_call(
        functools.partial(_mlp_body, D=D, Dh=Dh),
        grid=grid,
        in_specs=[
            pl.BlockSpec((R, D), lambda i: (i, 0)),
            pl.BlockSpec((R, Dh), lambda i: (i, 0)),
            pl.BlockSpec((R, Dh), lambda i: (i, 0)),
            pl.BlockSpec((IN, H), lambda i: (0, 0)),
            pl.BlockSpec((H, D), lambda i: (0, 0)),
            pl.BlockSpec((1, H), lambda i: (0, 0)),
            pl.BlockSpec((1, H), lambda i: (0, 0)),
            pl.BlockSpec((1, H), lambda i: (0, 0)),
            pl.BlockSpec((1, D), lambda i: (0, 0)),
        ],
        out_specs=pl.BlockSpec((R, D), lambda i: (i, 0)),
        out_shape=jax.ShapeDtypeStruct((N, D), jnp.float32),
    )(node_feat, a0, a1, W1, W2,
      b1.reshape(1, H), ln_g.reshape(1, H), ln_b.reshape(1, H), b2.reshape(1, D))
    return out


# speedup vs baseline: 4.0876x; 4.0876x over previous
import functools

import jax
import jax.numpy as jnp
from jax import lax
from jax.experimental import pallas as pl
from jax.experimental.pallas import tpu as pltpu
from jax.experimental.pallas import tpu_sc as plsc

CH = 40


def _make_aggregate(N, E, D, Np):
    Dh = D // 2
    n_sub = 16
    epw = E // n_sub
    nch = epw // CH
    rpt = Np // n_sub
    assert epw % CH == 0 and nch % 2 == 0 and Np % (8 * n_sub) == 0

    mesh = plsc.VectorSubcoreMesh(core_axis_name="c", subcore_axis_name="s")

    @functools.partial(
        pl.kernel,
        out_type=jax.ShapeDtypeStruct((2 * Np, Dh), jnp.float32),
        mesh=mesh,
        scratch_types=[
            pltpu.VMEM((nch, CH), jnp.int32),
            pltpu.VMEM((CH, Dh), jnp.float32),
            pltpu.VMEM((CH, Dh), jnp.float32),
            pltpu.VMEM_SHARED((Np, Dh), jnp.float32),
            pltpu.SemaphoreType.DMA,
            pltpu.SemaphoreType.DMA,
        ],
    )
    def agg(ea_hbm, dst_hbm, zeros_hbm, out_hbm, dst_v, rb0, rb1, acc, sem0, sem1):
        c = lax.axis_index("c")
        s = lax.axis_index("s")
        e_base = s * epw
        col = c * Dh

        pltpu.sync_copy(dst_hbm.at[s], dst_v)
        pltpu.sync_copy(zeros_hbm, acc.at[pl.ds(s * rpt, rpt), :])
        plsc.subcore_barrier()

        def rows_src(k):
            return ea_hbm.at[pl.ds(e_base + k * CH, CH), pl.ds(col, Dh)]

        pltpu.async_copy(rows_src(0), rb0, sem0)

        def body(j, carry):
            k = 2 * j
            pltpu.async_copy(rows_src(k + 1), rb1, sem1)
            pltpu.make_async_copy(rows_src(k), rb0, sem0).wait()
            pltpu.sync_copy(rb0, acc.at[dst_v.at[k]], add=True)

            @pl.when(j < nch // 2 - 1)
            def _():
                pltpu.async_copy(rows_src(k + 2), rb0, sem0)

            pltpu.make_async_copy(rows_src(k + 1), rb1, sem1).wait()
            pltpu.sync_copy(rb1, acc.at[dst_v.at[k + 1]], add=True)
            return carry

        lax.fori_loop(0, nch // 2, body, 0)
        plsc.subcore_barrier()

        pltpu.sync_copy(
            acc.at[pl.ds(s * rpt, rpt), :],
            out_hbm.at[pl.ds(c * Np + s * rpt, rpt), :],
        )

    return agg


def _mlp_body(nf, a0, a1, w1, w2, b1, g, b, b2, out, *, D, Dh):
    x = nf[...]
    h = jnp.dot(x, w1[0:D, :], preferred_element_type=jnp.float32)
    h = h + jnp.dot(a0[...], w1[D:D + Dh, :], preferred_element_type=jnp.float32)
    h = h + jnp.dot(a1[...], w1[D + Dh:, :], preferred_element_type=jnp.float32)
    h = h + b1[...]
    mu = jnp.mean(h, axis=-1, keepdims=True)
    var = jnp.mean((h - mu) ** 2, axis=-1, keepdims=True)
    hn = (h - mu) * lax.rsqrt(var + 1e-5) * g[...] + b[...]
    hs = hn * jax.nn.sigmoid(hn)
    out[...] = x + jnp.dot(hs, w2[...], preferred_element_type=jnp.float32) + b2[...]


def kernel(node_feat, edge_attr, edge_index, num_nodes, W1, b1, ln_g, ln_b, W2, b2):
    N, D = node_feat.shape
    E = edge_attr.shape[0]
    IN, H = W1.shape
    Dh = D // 2

    Np = ((N + 127) // 128) * 128
    dst3 = edge_index[1].reshape(16, E // (16 * CH), CH)
    zeros = jnp.zeros((Np // 16, Dh), jnp.float32)
    agg2 = _make_aggregate(N, E, D, Np)(edge_attr, dst3, zeros)
    a0, a1 = agg2[:N], agg2[Np:Np + N]

    R = 1000
    grid = (N // R,)
    out = pl.pallas_call(
        functools.partial(_mlp_body, D=D, Dh=Dh),
        grid=grid,
        in_specs=[
            pl.BlockSpec((R, D), lambda i: (i, 0)),
            pl.BlockSpec((R, Dh), lambda i: (i, 0)),
            pl.BlockSpec((R, Dh), lambda i: (i, 0)),
            pl.BlockSpec((IN, H), lambda i: (0, 0)),
            pl.BlockSpec((H, D), lambda i: (0, 0)),
            pl.BlockSpec((1, H), lambda i: (0, 0)),
            pl.BlockSpec((1, H), lambda i: (0, 0)),
            pl.BlockSpec((1, H), lambda i: (0, 0)),
            pl.BlockSpec((1, D), lambda i: (0, 0)),
        ],
        out_specs=pl.BlockSpec((R, D), lambda i: (i, 0)),
        out_shape=jax.ShapeDtypeStruct((N, D), jnp.float32),
    )(node_feat, a0, a1, W1, W2,
      b1.reshape(1, H), ln_g.reshape(1, H), ln_b.reshape(1, H), b2.reshape(1, D))
    return out
